# Initial kernel scaffold; baseline (speedup 1.0000x reference)
import jax
import jax.numpy as jnp
from jax.experimental import pallas as pl


def kernel(feature_matrix, edge_index, pairs_src, pairs_tgt, W_lin, b_lin, conv_W, Wd, bd, Wo, bo):
    raise NotImplementedError("write your pallas kernel here")


# speedup vs baseline: 2.5592x; 2.5592x over previous
import functools
import math

import jax
import jax.numpy as jnp
from jax import lax
from jax.experimental import pallas as pl
from jax.experimental.pallas import tpu as pltpu
from jax.experimental.pallas import tpu_sc as plsc

N = 10000
E = 320000
D = 256
H = 128
L = 4
P = 1024
ALPHA = 0.1
THETA = 0.5

NTILES = 16
EROWS = 2560
ROWS_PER_TILE = EROWS // NTILES
EPAD = EROWS * 128
DUMMY = N
SPAD = 10112
ICHUNK = 32
ROW_BLK = 400


def _in_proj_body(x_ref, w_ref, b_ref, out_ref):
    h = jnp.dot(x_ref[...], w_ref[...], preferred_element_type=jnp.float32, precision=lax.Precision.HIGHEST)
    h = jnp.maximum(h + b_ref[...], 0.0)
    out_ref[0] = h[:, :H]
    out_ref[1] = h[:, H:]


def _in_proj(x, w, b):
    grid = N // ROW_BLK
    return pl.pallas_call(
        _in_proj_body,
        grid=(grid,),
        in_specs=[
            pl.BlockSpec((ROW_BLK, D), lambda i: (i, 0)),
            pl.BlockSpec((D, D), lambda i: (0, 0)),
            pl.BlockSpec((1, D), lambda i: (0, 0)),
        ],
        out_specs=pl.BlockSpec((2, ROW_BLK, H), lambda i: (0, i, 0)),
        out_shape=jax.ShapeDtypeStruct((2, N, H), jnp.float32),
    )(x, w, b)


def _layer_body(beta, agg_ref, x0_ref, w_ref, out_ref):
    a = 1.0 - ALPHA
    s0 = a * agg_ref[0] + ALPHA * x0_ref[0]
    s1 = a * agg_ref[1] + ALPHA * x0_ref[1]
    s = jnp.concatenate([s0, s1], axis=1)
    m = jnp.dot(s, w_ref[...], preferred_element_type=jnp.float32, precision=lax.Precision.HIGHEST)
    o = jnp.maximum((1.0 - beta) * s + beta * m, 0.0)
    out_ref[0] = o[:, :H]
    out_ref[1] = o[:, H:]


def _layer(agg, x0, w, beta):
    grid = N // ROW_BLK
    return pl.pallas_call(
        functools.partial(_layer_body, beta),
        grid=(grid,),
        in_specs=[
            pl.BlockSpec((2, ROW_BLK, H), lambda i: (0, i, 0)),
            pl.BlockSpec((2, ROW_BLK, H), lambda i: (0, i, 0)),
            pl.BlockSpec((D, D), lambda i: (0, 0)),
        ],
        out_specs=pl.BlockSpec((2, ROW_BLK, H), lambda i: (0, i, 0)),
        out_shape=jax.ShapeDtypeStruct((2, N, H), jnp.float32),
    )(agg, x0, w)


def _cls_body(src_ref, tgt_ref, wd_ref, bd_ref, wo_ref, bo_ref,
              la_ref, pa_ref):
    se = jnp.tanh(jnp.dot(src_ref[...], wd_ref[...],
                          preferred_element_type=jnp.float32, precision=lax.Precision.HIGHEST) + bd_ref[...])
    te = jnp.tanh(jnp.dot(tgt_ref[...], wd_ref[...],
                          preferred_element_type=jnp.float32, precision=lax.Precision.HIGHEST) + bd_ref[...])
    feats = jnp.concatenate([se, te, jnp.abs(se - te), se * te], axis=1)
    logits = jnp.dot(feats, wo_ref[...],
                     preferred_element_type=jnp.float32, precision=lax.Precision.HIGHEST) + bo_ref[...]
    la_ref[...] = logits
    m = jnp.max(logits, axis=1, keepdims=True)
    e = jnp.exp(logits - m)
    pa_ref[...] = e / jnp.sum(e, axis=1, keepdims=True)


def _classifier(src_rows, tgt_rows, wd, bd, wo, bo):
    return pl.pallas_call(
        _cls_body,
        out_shape=(
            jax.ShapeDtypeStruct((P, 2), jnp.float32),
            jax.ShapeDtypeStruct((P, 2), jnp.float32),
        ),
    )(src_rows, tgt_rows, wd, bd, wo, bo)


def _seg_sum(h_cat, src2c, dst2):
    mesh = plsc.VectorSubcoreMesh(core_axis_name="c", subcore_axis_name="s")

    @functools.partial(
        pl.kernel, mesh=mesh,
        out_type=jax.ShapeDtypeStruct((2, N, H), jnp.float32),
        scratch_types=[
            pltpu.VMEM((ICHUNK, 128), jnp.int32),
            pltpu.VMEM((ICHUNK, 128), jnp.int32),
            pltpu.VMEM((128, H), jnp.float32),
            pltpu.VMEM_SHARED((SPAD, H), jnp.float32),
            pltpu.SemaphoreType.DMA,
        ],
    )
    def k(h_hbm, src_hbm, dst_hbm, out_hbm,
          src_v, dst_v, rows_v, acc_s, gsem):
        c = lax.axis_index("c")
        s = lax.axis_index("s")

        def zrow(i, _):
            for q in range(H // 16):
                rows_v[i, pl.ds(q * 16, 16)] = jnp.zeros((16,), jnp.float32)
            return 0
        lax.fori_loop(0, 128, zrow, 0, unroll=False)

        def zchunk(t, _):
            ch = s + t * NTILES

            @pl.when(ch < SPAD // 128)
            def _():
                pltpu.sync_copy(rows_v, acc_s.at[pl.ds(ch * 128, 128)])
            return 0
        lax.fori_loop(0, SPAD // 128 // NTILES + 1, zchunk, 0, unroll=False)
        plsc.subcore_barrier()

        def outer(t, _):
            base = s * ROWS_PER_TILE + t * ICHUNK
            pltpu.sync_copy(src_hbm.at[c, pl.ds(base, ICHUNK)], src_v)
            pltpu.sync_copy(dst_hbm.at[pl.ds(base, ICHUNK)], dst_v)

            def body(j, _):
                pltpu.async_copy(h_hbm.at[src_v.at[j]], rows_v, gsem).wait()
                pltpu.sync_copy(rows_v, acc_s.at[dst_v.at[j]], add=True)
                return 0
            lax.fori_loop(0, ICHUNK, body, 0, unroll=False)
            return 0
        lax.fori_loop(0, ROWS_PER_TILE // ICHUNK, outer, 0, unroll=False)
        plsc.subcore_barrier()

        @pl.when(s < NTILES - 1)
        def _():
            pltpu.sync_copy(acc_s.at[pl.ds(s * 624, 624)],
                            out_hbm.at[c, pl.ds(s * 624, 624)])

        @pl.when(s == NTILES - 1)
        def _():
            pltpu.sync_copy(acc_s.at[pl.ds(9360, 640)],
                            out_hbm.at[c, pl.ds(9360, 640)])

    return k(h_cat, src2c, dst2)


def _gather_rows(table, idx_flat):
    mesh = plsc.VectorSubcoreMesh(core_axis_name="c", subcore_axis_name="s")

    @functools.partial(
        pl.kernel, mesh=mesh,
        out_type=jax.ShapeDtypeStruct((4096, H), jnp.float32),
        scratch_types=[
            pltpu.VMEM((128,), jnp.int32),
            pltpu.VMEM((128, H), jnp.float32),
            pltpu.SemaphoreType.DMA,
        ],
    )
    def k(tab_hbm, idx_hbm, out_hbm, idx_v, rows_v, sem):
        c = lax.axis_index("c")
        s = lax.axis_index("s")
        wid = s * 2 + c
        pltpu.sync_copy(idx_hbm.at[pl.ds(wid * 128, 128)], idx_v)
        pltpu.async_copy(tab_hbm.at[idx_v], rows_v, sem).wait()
        pltpu.sync_copy(rows_v, out_hbm.at[pl.ds(wid * 128, 128)])

    return k(table, idx_flat)


def kernel(feature_matrix, edge_index, pairs_src, pairs_tgt,
           W_lin, b_lin, conv_W, Wd, bd, Wo, bo):
    src = edge_index[0]
    dst = edge_index[1]
    pad = EPAD - E
    srcp = jnp.concatenate([src, jnp.zeros((pad,), jnp.int32)])
    dstp = jnp.concatenate([dst, jnp.full((pad,), DUMMY, jnp.int32)])
    src2 = srcp.reshape(EROWS, 128)
    src2c = jnp.stack([src2, src2 + N])
    dst2 = dstp.reshape(EROWS, 128)

    h = _in_proj(feature_matrix, W_lin, b_lin.reshape(1, D))
    x0 = h
    for l in range(L):
        beta = float(math.log(THETA / (l + 1) + 1.0))
        agg = _seg_sum(h.reshape(2 * N, H), src2c, dst2)
        h = _layer(agg, x0, conv_W[l], beta)

    idx_all = jnp.concatenate([pairs_src, pairs_src + N,
                               pairs_tgt, pairs_tgt + N]).astype(jnp.int32)
    g = _gather_rows(h.reshape(2 * N, H), idx_all)
    src_rows = jnp.concatenate([g[0:P], g[P:2 * P]], axis=1)
    tgt_rows = jnp.concatenate([g[2 * P:3 * P], g[3 * P:]], axis=1)

    logits_all, probs_all = _classifier(src_rows, tgt_rows, Wd,
                                        bd.reshape(1, D), Wo, bo.reshape(1, 2))
    logits = logits_all[0:1]
    probs = probs_all[:, 1]
    src_embeds = probs_all[:, 0]
    tgt_embeds = probs_all[:, 1]
    return (logits, probs, src_embeds, tgt_embeds)
